# baseline (speedup 1.0000x reference)
import functools

import jax
import jax.numpy as jnp
from jax import lax
from jax.experimental import pallas as pl
from jax.experimental.pallas import tpu as pltpu
from jax.experimental.pallas import tpu_sc as plsc

_N = 1_000_000
_DIM = 64
_NC = 2
_NS = 16
_NW = _NC * _NS
_CHUNK = 128
_MAIN_ITERS = 244
_MAIN_PER_W = _MAIN_ITERS * _CHUNK
_MAIN = _NW * _MAIN_PER_W
_TAIL_CHUNK = 64
_TAIL_WORKERS = (_N - _MAIN) // _TAIL_CHUNK


@jax.jit
def _sc_gather(species, table):
    mesh = plsc.VectorSubcoreMesh(core_axis_name="c", subcore_axis_name="s")

    @functools.partial(
        pl.kernel,
        out_type=jax.ShapeDtypeStruct((_N, _DIM), jnp.float32),
        mesh=mesh,
        scratch_types=[
            pltpu.VMEM_SHARED((119, _DIM), jnp.float32),
            pltpu.VMEM((_CHUNK,), jnp.int32),
            pltpu.VMEM((_CHUNK, _DIM), jnp.float32),
            pltpu.VMEM((_TAIL_CHUNK,), jnp.int32),
            pltpu.VMEM((_TAIL_CHUNK, _DIM), jnp.float32),
            pltpu.SemaphoreType.DMA,
        ],
        compiler_params=pltpu.CompilerParams(use_tc_tiling_on_sc=False),
    )
    def k(species_hbm, table_hbm, out_hbm, table_sp, idx_v, rows_v, idx_t,
          rows_t, sem):
        wid = lax.axis_index("s") * _NC + lax.axis_index("c")
        base_w = wid * _MAIN_PER_W

        @pl.when(lax.axis_index("s") == 0)
        def _fill():
            pltpu.sync_copy(table_hbm, table_sp)

        plsc.subcore_barrier()

        def body(i, carry):
            base = base_w + i * _CHUNK
            pltpu.sync_copy(species_hbm.at[pl.ds(base, _CHUNK)], idx_v)
            pltpu.async_copy(table_sp.at[idx_v], rows_v, sem).wait()
            pltpu.sync_copy(rows_v, out_hbm.at[pl.ds(base, _CHUNK)])
            return carry

        lax.fori_loop(0, _MAIN_ITERS, body, 0)

        @pl.when(wid < _TAIL_WORKERS)
        def _tail():
            tb = _MAIN + wid * _TAIL_CHUNK
            pltpu.sync_copy(species_hbm.at[pl.ds(tb, _TAIL_CHUNK)], idx_t)
            pltpu.async_copy(table_sp.at[idx_t], rows_t, sem).wait()
            pltpu.sync_copy(rows_t, out_hbm.at[pl.ds(tb, _TAIL_CHUNK)])

    return k(species, table)


def kernel(species, conv_tensor):
    return _sc_gather(species, conv_tensor.astype(jnp.float32))
